# baseline (speedup 1.0000x reference)
from functools import partial

import jax
import jax.numpy as jnp
from jax import lax
from jax.experimental import pallas as pl
from jax.experimental.pallas import tpu as pltpu

_EPS = 1e-5


def _chan_stats(y):
    return jnp.concatenate([jnp.sum(y, axis=1, keepdims=True),
                            jnp.sum(y * y, axis=1, keepdims=True)], axis=1)


def _fold(stats, gamma_beta, count):
    mu = stats[:, 0:1] / count
    var = jnp.maximum(stats[:, 1:2] / count - mu * mu, 0.0)
    scale = gamma_beta[:, 0:1] * lax.rsqrt(var + _EPS)
    shift = gamma_beta[:, 1:2] - mu * scale
    return jnp.concatenate([scale, shift], axis=1)


def _fused_kernel(x_ref, w13l_ref, w13ce_ref, w2l_ref, w2c_ref, w2r_ref,
                  bn1_ref, bn2_ref, o_ref,
                  ybuf, st1_scr, st2_scr, ss1_scr, ss2_scr,
                  *, TN, C_in, C_half, C_out, L_out, count):
    p = pl.program_id(0)
    i = pl.program_id(1)

    @pl.when(jnp.logical_and(p == 0, i == 0))
    def _():
        st1_scr[...] = jnp.zeros_like(st1_scr)
        st2_scr[...] = jnp.zeros_like(st2_scr)

    @pl.when(p == 0)
    def _():
        w_l = w13l_ref[...]
        w_ce = w13ce_ref[...]
        lane = lax.broadcasted_iota(jnp.int32, (C_in, L_out), 1)
        not_first = lane > 0
        acc = jnp.zeros((C_half, 2), jnp.float32)
        for s in range(TN):
            xi = x_ref[s]
            xil = jnp.where(not_first, pltpu.roll(xi, shift=1, axis=1),
                            jnp.uint32(0))
            xb = pltpu.bitcast(xi, jnp.bfloat16)
            xlb = pltpu.bitcast(xil, jnp.bfloat16)
            y = (jnp.dot(w_ce, xb, preferred_element_type=jnp.float32)
                 + jnp.dot(w_l, xlb, preferred_element_type=jnp.float32))
            acc = acc + _chan_stats(y[:C_half, :])
            ybuf[i * TN + s] = y.astype(jnp.bfloat16)
        st1_scr[...] += acc

    @pl.when(jnp.logical_and(p == 1, i == 0))
    def _():
        ss1_scr[...] = _fold(st1_scr[...], bn1_ref[...], count)

    @pl.when(p == 1)
    def _():
        ss1 = ss1_scr[...]
        scale1, shift1 = ss1[:, 0:1], ss1[:, 1:2]
        w2l = w2l_ref[...]
        w2c = w2c_ref[...]
        w2r = w2r_ref[...]
        lane = lax.broadcasted_iota(jnp.int32, (C_half, L_out), 1)
        not_first = lane > 0
        not_last = lane < L_out - 1
        acc = jnp.zeros((C_out, 2), jnp.float32)
        for s in range(TN):
            y13 = ybuf[i * TN + s]
            a = y13[:C_half, :].astype(jnp.float32) * scale1 + shift1
            h = (a * jax.nn.sigmoid(a)).astype(jnp.bfloat16)
            h_l = jnp.where(not_first, pltpu.roll(h, shift=1, axis=1),
                            jnp.bfloat16(0))
            h_r = jnp.where(not_last, pltpu.roll(h, shift=L_out - 1, axis=1),
                            jnp.bfloat16(0))
            y2 = (jnp.dot(w2c, h, preferred_element_type=jnp.float32)
                  + jnp.dot(w2l, h_l, preferred_element_type=jnp.float32)
                  + jnp.dot(w2r, h_r, preferred_element_type=jnp.float32)
                  + y13[C_half:, :].astype(jnp.float32))
            acc = acc + _chan_stats(y2)
            ybuf[i * TN + s, :C_out, :] = y2.astype(jnp.bfloat16)
        st2_scr[...] += acc

    @pl.when(jnp.logical_and(p == 2, i == 0))
    def _():
        ss2_scr[...] = _fold(st2_scr[...], bn2_ref[...], count)

    @pl.when(p == 2)
    def _():
        ss2 = ss2_scr[...]
        scale2, shift2 = ss2[:, 0:1], ss2[:, 1:2]
        for s in range(TN):
            y2 = ybuf[i * TN + s, :C_out, :]
            a = y2.astype(jnp.float32) * scale2 + shift2
            o_ref[s] = a * jax.nn.sigmoid(a)


def kernel(x, w1, b1, g1, be1, w2, b2, g2, be2, w3, b3):
    N, C_in, L = x.shape
    C_half = w1.shape[0]
    C_out = w2.shape[0]
    L_out = (L + 1) // 2
    C13 = C_half + C_out
    count = float(N * L_out)

    xb = x.astype(jnp.bfloat16)
    if L % 2:
        xb = jnp.pad(xb, ((0, 0), (0, 0), (0, 1)))
    xp = lax.bitcast_convert_type(xb.reshape(N, C_in, L_out, 2), jnp.uint32)

    w13 = jnp.concatenate([w1, w3], axis=0)
    w13l = jnp.stack([jnp.zeros_like(w13[:, :, 0]), w13[:, :, 0]],
                     axis=2).reshape(C13, 2 * C_in).astype(jnp.bfloat16)
    w13ce = jnp.stack([w13[:, :, 1], w13[:, :, 2]],
                      axis=2).reshape(C13, 2 * C_in).astype(jnp.bfloat16)
    w2l = w2[:, :, 0].astype(jnp.bfloat16)
    w2c = w2[:, :, 1].astype(jnp.bfloat16)
    w2r = w2[:, :, 2].astype(jnp.bfloat16)
    bn1p = jnp.stack([g1, be1], axis=1).astype(jnp.float32)
    bn2p = jnp.stack([g2, be2], axis=1).astype(jnp.float32)

    TN = 1
    for d in range(1, min(N, 16) + 1):
        if N % d == 0:
            TN = d
    n_tiles = N // TN

    return pl.pallas_call(
        partial(_fused_kernel, TN=TN, C_in=C_in, C_half=C_half, C_out=C_out,
                L_out=L_out, count=count),
        grid=(3, n_tiles),
        in_specs=[
            pl.BlockSpec((TN, C_in, L_out),
                         lambda p, i: (i * ((2 - p) // 2), 0, 0)),
            pl.BlockSpec((C13, 2 * C_in), lambda p, i: (0, 0)),
            pl.BlockSpec((C13, 2 * C_in), lambda p, i: (0, 0)),
            pl.BlockSpec((C_out, C_half), lambda p, i: (0, 0)),
            pl.BlockSpec((C_out, C_half), lambda p, i: (0, 0)),
            pl.BlockSpec((C_out, C_half), lambda p, i: (0, 0)),
            pl.BlockSpec((C_half, 2), lambda p, i: (0, 0)),
            pl.BlockSpec((C_out, 2), lambda p, i: (0, 0)),
        ],
        out_specs=pl.BlockSpec((TN, C_out, L_out),
                               lambda p, i: (i * (p // 2), 0, 0)),
        out_shape=jax.ShapeDtypeStruct((N, C_out, L_out), jnp.float32),
        scratch_shapes=[
            pltpu.VMEM((N, C13, L_out), jnp.bfloat16),
            pltpu.VMEM((C_half, 2), jnp.float32),
            pltpu.VMEM((C_out, 2), jnp.float32),
            pltpu.VMEM((C_half, 2), jnp.float32),
            pltpu.VMEM((C_out, 2), jnp.float32),
        ],
        compiler_params=pltpu.CompilerParams(
            dimension_semantics=("arbitrary", "arbitrary"),
            vmem_limit_bytes=64 * 2**20),
    )(xp, w13l, w13ce, w2l, w2c, w2r, bn1p, bn2p)


# speedup vs baseline: 1.5130x; 1.5130x over previous
from functools import partial

import jax
import jax.numpy as jnp
from jax import lax
from jax.experimental import pallas as pl
from jax.experimental.pallas import tpu as pltpu

_EPS = 1e-5


def _chan_stats(y):
    return jnp.concatenate([jnp.sum(y, axis=1, keepdims=True),
                            jnp.sum(y * y, axis=1, keepdims=True)], axis=1)


def _fold(stats, gamma_beta, count):
    mu = stats[:, 0:1] / count
    var = jnp.maximum(stats[:, 1:2] / count - mu * mu, 0.0)
    scale = gamma_beta[:, 0:1] * lax.rsqrt(var + _EPS)
    shift = gamma_beta[:, 1:2] - mu * scale
    return jnp.concatenate([scale, shift], axis=1)


def _fused_kernel(x_ref, sel_ref, w13l_ref, w13c_ref, w13r_ref,
                  w2l_ref, w2c_ref, w2r_ref,
                  bn1_ref, bn2_ref, o_ref,
                  ybuf, st1_scr, st2_scr, ss1_scr, ss2_scr,
                  *, TN, C_in, C_half, C_out, L_out, count):
    p = pl.program_id(0)
    i = pl.program_id(1)

    @pl.when(jnp.logical_and(p == 0, i == 0))
    def _():
        st1_scr[...] = jnp.zeros_like(st1_scr)
        st2_scr[...] = jnp.zeros_like(st2_scr)

    @pl.when(p == 0)
    def _():
        w_l = w13l_ref[...]
        w_c = w13c_ref[...]
        w_r = w13r_ref[...]
        s_e = sel_ref[:, :L_out]
        s_o = sel_ref[:, L_out:2 * L_out]
        s_l = sel_ref[:, 2 * L_out:]
        acc = jnp.zeros((C_half, 2), jnp.float32)
        for s in range(TN):
            xb = x_ref[s].astype(jnp.bfloat16)
            xe = jnp.dot(xb, s_e,
                         preferred_element_type=jnp.float32).astype(jnp.bfloat16)
            xo = jnp.dot(xb, s_o,
                         preferred_element_type=jnp.float32).astype(jnp.bfloat16)
            xl = jnp.dot(xb, s_l,
                         preferred_element_type=jnp.float32).astype(jnp.bfloat16)
            y = (jnp.dot(w_c, xe, preferred_element_type=jnp.float32)
                 + jnp.dot(w_r, xo, preferred_element_type=jnp.float32)
                 + jnp.dot(w_l, xl, preferred_element_type=jnp.float32))
            acc = acc + _chan_stats(y[:C_half, :])
            ybuf[i * TN + s] = y.astype(jnp.bfloat16)
        st1_scr[...] += acc

    @pl.when(jnp.logical_and(p == 1, i == 0))
    def _():
        ss1_scr[...] = _fold(st1_scr[...], bn1_ref[...], count)

    @pl.when(p == 1)
    def _():
        ss1 = ss1_scr[...]
        scale1, shift1 = ss1[:, 0:1], ss1[:, 1:2]
        w2l = w2l_ref[...]
        w2c = w2c_ref[...]
        w2r = w2r_ref[...]
        lane = lax.broadcasted_iota(jnp.int32, (C_half, L_out), 1)
        not_first = lane > 0
        not_last = lane < L_out - 1
        acc = jnp.zeros((C_out, 2), jnp.float32)
        for s in range(TN):
            y13 = ybuf[i * TN + s]
            a = y13[:C_half, :].astype(jnp.float32) * scale1 + shift1
            h = (a * jax.nn.sigmoid(a)).astype(jnp.bfloat16)
            h_l = jnp.where(not_first, pltpu.roll(h, shift=1, axis=1),
                            jnp.bfloat16(0))
            h_r = jnp.where(not_last, pltpu.roll(h, shift=L_out - 1, axis=1),
                            jnp.bfloat16(0))
            y2 = (jnp.dot(w2c, h, preferred_element_type=jnp.float32)
                  + jnp.dot(w2l, h_l, preferred_element_type=jnp.float32)
                  + jnp.dot(w2r, h_r, preferred_element_type=jnp.float32)
                  + y13[C_half:, :].astype(jnp.float32))
            acc = acc + _chan_stats(y2)
            ybuf[i * TN + s, :C_out, :] = y2.astype(jnp.bfloat16)
        st2_scr[...] += acc

    @pl.when(jnp.logical_and(p == 2, i == 0))
    def _():
        ss2_scr[...] = _fold(st2_scr[...], bn2_ref[...], count)

    @pl.when(p == 2)
    def _():
        ss2 = ss2_scr[...]
        scale2, shift2 = ss2[:, 0:1], ss2[:, 1:2]
        for s in range(TN):
            y2 = ybuf[i * TN + s, :C_out, :]
            a = y2.astype(jnp.float32) * scale2 + shift2
            o_ref[s] = a * jax.nn.sigmoid(a)


def kernel(x, w1, b1, g1, be1, w2, b2, g2, be2, w3, b3):
    N, C_in, L = x.shape
    C_half = w1.shape[0]
    C_out = w2.shape[0]
    L_out = (L + 1) // 2
    C13 = C_half + C_out
    Lp = 2 * L_out
    count = float(N * L_out)

    xf = x.astype(jnp.float32)
    if L % 2:
        xf = jnp.pad(xf, ((0, 0), (0, 0), (0, 1)))

    pos = jnp.arange(Lp, dtype=jnp.int32)[:, None]
    t2 = 2 * jnp.arange(L_out, dtype=jnp.int32)[None, :]
    sel = jnp.concatenate([(pos == t2), (pos == t2 + 1), (pos == t2 - 1)],
                          axis=1).astype(jnp.bfloat16)

    w13 = jnp.concatenate([w1, w3], axis=0)
    w13l = w13[:, :, 0].astype(jnp.bfloat16)
    w13c = w13[:, :, 1].astype(jnp.bfloat16)
    w13r = w13[:, :, 2].astype(jnp.bfloat16)
    w2l = w2[:, :, 0].astype(jnp.bfloat16)
    w2c = w2[:, :, 1].astype(jnp.bfloat16)
    w2r = w2[:, :, 2].astype(jnp.bfloat16)
    bn1p = jnp.stack([g1, be1], axis=1).astype(jnp.float32)
    bn2p = jnp.stack([g2, be2], axis=1).astype(jnp.float32)

    TN = 1
    for d in range(1, min(N, 8) + 1):
        if N % d == 0:
            TN = d
    n_tiles = N // TN

    return pl.pallas_call(
        partial(_fused_kernel, TN=TN, C_in=C_in, C_half=C_half, C_out=C_out,
                L_out=L_out, count=count),
        grid=(3, n_tiles),
        in_specs=[
            pl.BlockSpec((TN, C_in, Lp),
                         lambda p, i: (i * ((2 - p) // 2), 0, 0)),
            pl.BlockSpec((Lp, 3 * L_out), lambda p, i: (0, 0)),
            pl.BlockSpec((C13, C_in), lambda p, i: (0, 0)),
            pl.BlockSpec((C13, C_in), lambda p, i: (0, 0)),
            pl.BlockSpec((C13, C_in), lambda p, i: (0, 0)),
            pl.BlockSpec((C_out, C_half), lambda p, i: (0, 0)),
            pl.BlockSpec((C_out, C_half), lambda p, i: (0, 0)),
            pl.BlockSpec((C_out, C_half), lambda p, i: (0, 0)),
            pl.BlockSpec((C_half, 2), lambda p, i: (0, 0)),
            pl.BlockSpec((C_out, 2), lambda p, i: (0, 0)),
        ],
        out_specs=pl.BlockSpec((TN, C_out, L_out),
                               lambda p, i: (i * (p // 2), 0, 0)),
        out_shape=jax.ShapeDtypeStruct((N, C_out, L_out), jnp.float32),
        scratch_shapes=[
            pltpu.VMEM((N, C13, L_out), jnp.bfloat16),
            pltpu.VMEM((C_half, 2), jnp.float32),
            pltpu.VMEM((C_out, 2), jnp.float32),
            pltpu.VMEM((C_half, 2), jnp.float32),
            pltpu.VMEM((C_out, 2), jnp.float32),
        ],
        compiler_params=pltpu.CompilerParams(
            dimension_semantics=("arbitrary", "arbitrary"),
            vmem_limit_bytes=64 * 2**20),
    )(xf, sel, w13l, w13c, w13r, w2l, w2c, w2r, bn1p, bn2p)


# speedup vs baseline: 1.7828x; 1.1783x over previous
from functools import partial

import jax
import jax.numpy as jnp
from jax import lax
from jax.experimental import pallas as pl
from jax.experimental.pallas import tpu as pltpu

_EPS = 1e-5


def _chan_stats(y):
    return jnp.concatenate([jnp.sum(y, axis=1, keepdims=True),
                            jnp.sum(y * y, axis=1, keepdims=True)], axis=1)


def _fold(stats, gamma_beta, count):
    mu = stats[:, 0:1] / count
    var = jnp.maximum(stats[:, 1:2] / count - mu * mu, 0.0)
    scale = gamma_beta[:, 0:1] * lax.rsqrt(var + _EPS)
    shift = gamma_beta[:, 1:2] - mu * scale
    return jnp.concatenate([scale, shift], axis=1)


def _fused_kernel(x_ref, sel_ref, w13l_ref, w13ce_ref,
                  w2l_ref, w2c_ref, w2r_ref,
                  bn1_ref, bn2_ref, o_ref,
                  ybuf, st1_scr, st2_scr, ss1_scr, ss2_scr,
                  *, TN, C_in, C_half, C_out, L_out, count):
    p = pl.program_id(0)
    i = pl.program_id(1)

    @pl.when(jnp.logical_and(p == 0, i == 0))
    def _():
        st1_scr[...] = jnp.zeros_like(st1_scr)
        st2_scr[...] = jnp.zeros_like(st2_scr)

    @pl.when(p == 0)
    def _():
        w_l = w13l_ref[...]
        w_ce = w13ce_ref[...]
        b_eo = sel_ref[...]
        lane = lax.broadcasted_iota(jnp.int32, (C_in, L_out), 1)
        not_first = lane > 0
        CH = (2 * L_out) // 256
        acc = jnp.zeros((C_half, 2), jnp.float32)
        for s in range(TN):
            us = [jnp.dot(x_ref[s, :, 256 * j:256 * (j + 1)]
                          .astype(jnp.bfloat16), b_eo,
                          preferred_element_type=jnp.float32)
                  for j in range(CH)]
            xe = jnp.concatenate([u[:, :128] for u in us],
                                 axis=1).astype(jnp.bfloat16)
            xo = jnp.concatenate([u[:, 128:] for u in us],
                                 axis=1).astype(jnp.bfloat16)
            xl = jnp.where(not_first, pltpu.roll(xo, shift=1, axis=1),
                           jnp.bfloat16(0))
            x_eo = jnp.concatenate([xe, xo], axis=0)
            y = (jnp.dot(w_ce, x_eo, preferred_element_type=jnp.float32)
                 + jnp.dot(w_l, xl, preferred_element_type=jnp.float32))
            acc = acc + _chan_stats(y[:C_half, :])
            ybuf[i * TN + s] = y.astype(jnp.bfloat16)
        st1_scr[...] += acc

    @pl.when(jnp.logical_and(p == 1, i == 0))
    def _():
        ss1_scr[...] = _fold(st1_scr[...], bn1_ref[...], count)

    @pl.when(p == 1)
    def _():
        ss1 = ss1_scr[...]
        scale1, shift1 = ss1[:, 0:1], ss1[:, 1:2]
        w2l = w2l_ref[...]
        w2c = w2c_ref[...]
        w2r = w2r_ref[...]
        lane = lax.broadcasted_iota(jnp.int32, (C_half, L_out), 1)
        not_first = lane > 0
        not_last = lane < L_out - 1
        acc = jnp.zeros((C_out, 2), jnp.float32)
        for s in range(TN):
            y13 = ybuf[i * TN + s]
            a = y13[:C_half, :].astype(jnp.float32) * scale1 + shift1
            h = (a * jax.nn.sigmoid(a)).astype(jnp.bfloat16)
            h_l = jnp.where(not_first, pltpu.roll(h, shift=1, axis=1),
                            jnp.bfloat16(0))
            h_r = jnp.where(not_last, pltpu.roll(h, shift=L_out - 1, axis=1),
                            jnp.bfloat16(0))
            y2 = (jnp.dot(w2c, h, preferred_element_type=jnp.float32)
                  + jnp.dot(w2l, h_l, preferred_element_type=jnp.float32)
                  + jnp.dot(w2r, h_r, preferred_element_type=jnp.float32)
                  + y13[C_half:, :].astype(jnp.float32))
            acc = acc + _chan_stats(y2)
            ybuf[i * TN + s, :C_out, :] = y2.astype(jnp.bfloat16)
        st2_scr[...] += acc

    @pl.when(jnp.logical_and(p == 2, i == 0))
    def _():
        ss2_scr[...] = _fold(st2_scr[...], bn2_ref[...], count)

    @pl.when(p == 2)
    def _():
        ss2 = ss2_scr[...]
        scale2, shift2 = ss2[:, 0:1], ss2[:, 1:2]
        for s in range(TN):
            y2 = ybuf[i * TN + s, :C_out, :]
            a = y2.astype(jnp.float32) * scale2 + shift2
            o_ref[s] = a * jax.nn.sigmoid(a)


def kernel(x, w1, b1, g1, be1, w2, b2, g2, be2, w3, b3):
    N, C_in, L = x.shape
    C_half = w1.shape[0]
    C_out = w2.shape[0]
    L_out = (L + 1) // 2
    C13 = C_half + C_out
    Lp = 2 * L_out
    count = float(N * L_out)

    xf = x.astype(jnp.float32)
    if L % 2:
        xf = jnp.pad(xf, ((0, 0), (0, 0), (0, 1)))

    pos = jnp.arange(256, dtype=jnp.int32)[:, None]
    t2 = 2 * jnp.arange(128, dtype=jnp.int32)[None, :]
    sel = jnp.concatenate([(pos == t2), (pos == t2 + 1)],
                          axis=1).astype(jnp.bfloat16)

    w13 = jnp.concatenate([w1, w3], axis=0)
    w13l = w13[:, :, 0].astype(jnp.bfloat16)
    w13ce = jnp.concatenate([w13[:, :, 1], w13[:, :, 2]],
                            axis=1).astype(jnp.bfloat16)
    w2l = w2[:, :, 0].astype(jnp.bfloat16)
    w2c = w2[:, :, 1].astype(jnp.bfloat16)
    w2r = w2[:, :, 2].astype(jnp.bfloat16)
    bn1p = jnp.stack([g1, be1], axis=1).astype(jnp.float32)
    bn2p = jnp.stack([g2, be2], axis=1).astype(jnp.float32)

    TN = 1
    for d in range(1, min(N, 8) + 1):
        if N % d == 0:
            TN = d
    n_tiles = N // TN

    return pl.pallas_call(
        partial(_fused_kernel, TN=TN, C_in=C_in, C_half=C_half, C_out=C_out,
                L_out=L_out, count=count),
        grid=(3, n_tiles),
        in_specs=[
            pl.BlockSpec((TN, C_in, Lp),
                         lambda p, i: (i * ((2 - p) // 2), 0, 0)),
            pl.BlockSpec((256, 256), lambda p, i: (0, 0)),
            pl.BlockSpec((C13, C_in), lambda p, i: (0, 0)),
            pl.BlockSpec((C13, 2 * C_in), lambda p, i: (0, 0)),
            pl.BlockSpec((C_out, C_half), lambda p, i: (0, 0)),
            pl.BlockSpec((C_out, C_half), lambda p, i: (0, 0)),
            pl.BlockSpec((C_out, C_half), lambda p, i: (0, 0)),
            pl.BlockSpec((C_half, 2), lambda p, i: (0, 0)),
            pl.BlockSpec((C_out, 2), lambda p, i: (0, 0)),
        ],
        out_specs=pl.BlockSpec((TN, C_out, L_out),
                               lambda p, i: (i * (p // 2), 0, 0)),
        out_shape=jax.ShapeDtypeStruct((N, C_out, L_out), jnp.float32),
        scratch_shapes=[
            pltpu.VMEM((N, C13, L_out), jnp.bfloat16),
            pltpu.VMEM((C_half, 2), jnp.float32),
            pltpu.VMEM((C_out, 2), jnp.float32),
            pltpu.VMEM((C_half, 2), jnp.float32),
            pltpu.VMEM((C_out, 2), jnp.float32),
        ],
        compiler_params=pltpu.CompilerParams(
            dimension_semantics=("arbitrary", "arbitrary"),
            vmem_limit_bytes=64 * 2**20),
    )(xf, sel, w13l, w13ce, w2l, w2c, w2r, bn1p, bn2p)
